# baseline (speedup 1.0000x reference)
import functools

import jax
import jax.numpy as jnp
from jax import lax
from jax.experimental import pallas as pl
from jax.experimental.pallas import tpu as pltpu
from jax.experimental.pallas import tpu_sc as plsc

N_CORES = 2
N_SUBCORES = 16
LANES = 16
N_WORKERS = N_CORES * N_SUBCORES

BATCH = 16384
D = 64
PER_W = BATCH // N_WORKERS
DMA_CHUNK = 128
N_CHUNKS = PER_W // DMA_CHUNK
N_GROUPS = PER_W // LANES


def _sc_body(uidx_hbm, iidx_hbm, uf_hbm, if_hbm, ub_hbm, ib_hbm, out_hbm,
             uidx_v, iidx_v, urow_v, irow_v, ubias_v, ibias_v, out_v, sem):
    wid = lax.axis_index("s") * N_CORES + lax.axis_index("c")
    idx_base = wid * N_CHUNKS

    pltpu.sync_copy(uidx_hbm.at[pl.ds(idx_base, N_CHUNKS)], uidx_v)
    pltpu.sync_copy(iidx_hbm.at[pl.ds(idx_base, N_CHUNKS)], iidx_v)

    copies = []
    for j in range(N_CHUNKS):
        rows = pl.ds(j * DMA_CHUNK, DMA_CHUNK)
        copies.append(pltpu.async_copy(uf_hbm.at[uidx_v.at[j]], urow_v.at[rows], sem))
        copies.append(pltpu.async_copy(if_hbm.at[iidx_v.at[j]], irow_v.at[rows], sem))
        copies.append(pltpu.async_copy(ub_hbm.at[uidx_v.at[j]], ubias_v.at[rows], sem))
        copies.append(pltpu.async_copy(ib_hbm.at[iidx_v.at[j]], ibias_v.at[rows], sem))
    for cp in copies:
        cp.wait()

    lane = lax.iota(jnp.int32, LANES)
    zeros = lane * 0

    def group(g, carry):
        rows16 = g * LANES + lane
        acc = ubias_v[pl.ds(g * LANES, LANES)] + ibias_v[pl.ds(g * LANES, LANES)]
        for j in range(D):
            colj = zeros + j
            u = plsc.load_gather(urow_v, [rows16, colj])
            v = plsc.load_gather(irow_v, [rows16, colj])
            acc = acc + u * v
        acc = 1.0 / (1.0 + jnp.exp(-acc))
        out_v[pl.ds(g * LANES, LANES)] = acc
        return carry

    lax.fori_loop(0, N_GROUPS, group, 0)
    pltpu.sync_copy(out_v, out_hbm.at[pl.ds(wid * PER_W, PER_W)])


@jax.jit
def _baseline_cf_sc(uidx, iidx, user_factors, item_factors, user_bias, item_bias):
    mesh = plsc.VectorSubcoreMesh(core_axis_name="c", subcore_axis_name="s")
    run = functools.partial(
        pl.kernel,
        mesh=mesh,
        compiler_params=pltpu.CompilerParams(
            needs_layout_passes=False, use_tc_tiling_on_sc=False),
        out_type=jax.ShapeDtypeStruct((BATCH,), jnp.float32),
        scratch_types=[
            pltpu.VMEM((N_CHUNKS, DMA_CHUNK), jnp.int32),
            pltpu.VMEM((N_CHUNKS, DMA_CHUNK), jnp.int32),
            pltpu.VMEM((PER_W, D), jnp.float32),
            pltpu.VMEM((PER_W, D), jnp.float32),
            pltpu.VMEM((PER_W,), jnp.float32),
            pltpu.VMEM((PER_W,), jnp.float32),
            pltpu.VMEM((PER_W,), jnp.float32),
            pltpu.SemaphoreType.DMA,
        ],
    )(_sc_body)
    return run(uidx, iidx, user_factors, item_factors, user_bias, item_bias)


def kernel(data, user_factors, item_factors, user_bias, item_bias):
    uidx = data[:, 0].reshape(N_WORKERS * N_CHUNKS, DMA_CHUNK)
    iidx = data[:, 1].reshape(N_WORKERS * N_CHUNKS, DMA_CHUNK)
    out = _baseline_cf_sc(uidx, iidx, user_factors, item_factors,
                          user_bias.reshape(-1), item_bias.reshape(-1))
    return out.reshape(BATCH, 1)


# speedup vs baseline: 2.1079x; 2.1079x over previous
import functools

import jax
import jax.numpy as jnp
from jax import lax
from jax.experimental import pallas as pl
from jax.experimental.pallas import tpu as pltpu
from jax.experimental.pallas import tpu_sc as plsc

N_CORES = 2
N_SUBCORES = 16
LANES = 16
N_WORKERS = N_CORES * N_SUBCORES

BATCH = 16384
D = 64
PER_W = BATCH // N_WORKERS
CH = 32
N_CH = PER_W // CH
G_PER_CH = CH // LANES


def _sc_body(uidx_hbm, iidx_hbm, uf_hbm, if_hbm, ub_hbm, ib_hbm, dummy_hbm,
             out_hbm, uidx_v, iidx_v, utile_v, itile_v,
             ubias_v, ibias_v, out_v, sem_u, sem_i, sem_b):
    wid = lax.axis_index("s") * N_CORES + lax.axis_index("c")

    pltpu.sync_copy(uidx_hbm.at[pl.ds(wid, 1)], uidx_v)
    pltpu.sync_copy(iidx_hbm.at[pl.ds(wid, 1)], iidx_v)

    def fire_bias(g, carry):
        uvals = uidx_v[0, pl.ds(g * LANES, LANES)]
        ivals = iidx_v[0, pl.ds(g * LANES, LANES)]
        for p in range(LANES):
            su = uvals[p]
            si = ivals[p]
            su_al = pl.multiple_of(su & ~7, 8)
            si_al = pl.multiple_of(si & ~7, 8)
            pb = pl.multiple_of((g * LANES + p) * 8, 8)
            pltpu.make_async_copy(
                ub_hbm.at[pl.ds(su_al, 8)], ubias_v.at[pl.ds(pb, 8)],
                sem_b).start()
            pltpu.make_async_copy(
                ib_hbm.at[pl.ds(si_al, 8)], ibias_v.at[pl.ds(pb, 8)],
                sem_b).start()
        return carry

    lax.fori_loop(0, PER_W // LANES, fire_bias, 0)

    lane = lax.iota(jnp.int32, LANES)
    zeros = lane * 0

    def chunk(c, carry):
        for g in range(G_PER_CH):
            uvals = uidx_v[0, pl.ds(c * CH + g * LANES, LANES)]
            ivals = iidx_v[0, pl.ds(c * CH + g * LANES, LANES)]
            for p in range(LANES):
                su = uvals[p]
                si = ivals[p]
                pltpu.make_async_copy(
                    uf_hbm.at[pl.ds(su >> 3, 1)],
                    utile_v.at[pl.ds(g * LANES + p, 1)], sem_u).start()
                pltpu.make_async_copy(
                    if_hbm.at[pl.ds(si >> 3, 1)],
                    itile_v.at[pl.ds(g * LANES + p, 1)], sem_i).start()
        pltpu.make_async_copy(dummy_hbm, utile_v, sem_u).wait()
        pltpu.make_async_copy(dummy_hbm, itile_v, sem_i).wait()

        for g in range(G_PER_CH):
            p_loc = g * LANES + lane
            rows16 = c * CH + p_loc
            uvals = uidx_v[0, pl.ds(c * CH + g * LANES, LANES)]
            ivals = iidx_v[0, pl.ds(c * CH + g * LANES, LANES)]
            urow = uvals & 7
            irow = ivals & 7
            acc = plsc.load_gather(ubias_v, [rows16 * 8 + urow])
            acc = acc + plsc.load_gather(ibias_v, [rows16 * 8 + irow])
            for j in range(D):
                colj = zeros + j
                u = plsc.load_gather(utile_v, [p_loc, urow, colj])
                v = plsc.load_gather(itile_v, [p_loc, irow, colj])
                acc = acc + u * v
            acc = 1.0 / (1.0 + jnp.exp(-acc))
            out_v[pl.ds(c * CH + g * LANES, LANES)] = acc
        return carry

    lax.fori_loop(0, N_CH, chunk, 0)
    pltpu.sync_copy(out_v, out_hbm.at[pl.ds(wid * PER_W, PER_W)])


@jax.jit
def _baseline_cf_sc(uidx, iidx, uf3, if3, user_bias, item_bias, dummy):
    mesh = plsc.VectorSubcoreMesh(core_axis_name="c", subcore_axis_name="s")
    run = functools.partial(
        pl.kernel,
        mesh=mesh,
        compiler_params=pltpu.CompilerParams(needs_layout_passes=False),
        out_type=jax.ShapeDtypeStruct((BATCH,), jnp.float32),
        scratch_types=[
            pltpu.VMEM((1, PER_W), jnp.int32),
            pltpu.VMEM((1, PER_W), jnp.int32),
            pltpu.VMEM((CH, 8, D), jnp.float32),
            pltpu.VMEM((CH, 8, D), jnp.float32),
            pltpu.VMEM((PER_W * 8,), jnp.float32),
            pltpu.VMEM((PER_W * 8,), jnp.float32),
            pltpu.VMEM((PER_W,), jnp.float32),
            pltpu.SemaphoreType.DMA,
            pltpu.SemaphoreType.DMA,
            pltpu.SemaphoreType.DMA,
        ],
    )(_sc_body)
    return run(uidx, iidx, uf3, if3, user_bias, item_bias, dummy)


def kernel(data, user_factors, item_factors, user_bias, item_bias):
    uidx = data[:, 0].reshape(N_WORKERS, PER_W)
    iidx = data[:, 1].reshape(N_WORKERS, PER_W)
    uf3 = user_factors.reshape(125000, 8, D)
    if3 = item_factors.reshape(125000, 8, D)
    dummy = jnp.zeros((CH, 8, D), jnp.float32)
    out = _baseline_cf_sc(uidx, iidx, uf3, if3,
                          user_bias.reshape(-1), item_bias.reshape(-1), dummy)
    return out.reshape(BATCH, 1)
